# Initial kernel scaffold; baseline (speedup 1.0000x reference)
import jax
import jax.numpy as jnp
from jax.experimental import pallas as pl


def kernel(inputs, token_table, pos_table):
    raise NotImplementedError("write your pallas kernel here")


# speedup vs baseline: 1.2466x; 1.2466x over previous
import functools

import jax
import jax.numpy as jnp
from jax import lax
from jax.experimental import pallas as pl
from jax.experimental.pallas import tpu as pltpu
from jax.experimental.pallas import tpu_sc as plsc

NC = 2
NS = 16
NW = NC * NS
LANES = 16
IDX_PER_DMA = 128
CHUNK = 256
NBUF = 4
ADD_UNROLL = 4


@functools.partial(jax.jit, static_argnames=("seq", "dim", "rows_per_w"))
def _embed(idx3, token_table, pos_table, *, seq, dim, rows_per_w):
    total = NW * rows_per_w
    nchunk = rows_per_w // CHUNK
    dmas_per_chunk = CHUNK // IDX_PER_DMA
    nrep = (CHUNK - 1 + seq) // seq + 1

    def body(idx_hbm, tok_hbm, pos_hbm, out_hbm, idx_v, pos_v,
             rows0, rows1, rows2, rows3, gsem, osem):
        bufs = (rows0, rows1, rows2, rows3)
        wid = lax.axis_index("s") * NC + lax.axis_index("c")
        base_row = wid * rows_per_w

        pltpu.sync_copy(idx_hbm.at[wid], idx_v)
        for r in range(nrep):
            pltpu.sync_copy(pos_hbm, pos_v.at[pl.ds(r * seq, seq)])

        def start_gather(g, b):
            for d in range(dmas_per_chunk):
                pltpu.async_copy(
                    tok_hbm.at[idx_v.at[dmas_per_chunk * g + d]],
                    bufs[b].at[pl.ds(d * IDX_PER_DMA, IDX_PER_DMA)],
                    gsem.at[b])

        def wait_gather(g, b):
            for d in range(dmas_per_chunk):
                pltpu.make_async_copy(
                    tok_hbm.at[idx_v.at[dmas_per_chunk * g + d]],
                    bufs[b].at[pl.ds(d * IDX_PER_DMA, IDX_PER_DMA)],
                    gsem.at[b]).wait()

        def start_out(g, b):
            pltpu.async_copy(
                bufs[b], out_hbm.at[pl.ds(base_row + g * CHUNK, CHUNK)],
                osem.at[b])

        def wait_out(g, b):
            pltpu.make_async_copy(
                bufs[b], out_hbm.at[pl.ds(base_row + g * CHUNK, CHUNK)],
                osem.at[b]).wait()

        def add_pos(g, b):
            phase = lax.rem(g * CHUNK, seq)
            rows = bufs[b]

            def jbody(jj, carry):
                for u in range(ADD_UNROLL):
                    j = jj * ADD_UNROLL + u
                    p = phase + j
                    for h in range(0, dim, LANES):
                        rows[j, pl.ds(h, LANES)] = (
                            rows[j, pl.ds(h, LANES)]
                            + pos_v[p, pl.ds(h, LANES)])
                return carry

            lax.fori_loop(0, CHUNK // ADD_UNROLL, jbody, 0)

        start_gather(0, 0)

        def gg_body(gg, carry):
            for b in range(NBUF):
                g = gg * NBUF + b
                nb = (b + 1) % NBUF

                @pl.when(g >= NBUF - 1)
                def _():
                    wait_out(g - (NBUF - 1), nb)

                @pl.when(g + 1 < nchunk)
                def _():
                    start_gather(g + 1, nb)

                wait_gather(g, b)
                add_pos(g, b)
                start_out(g, b)
            return carry

        lax.fori_loop(0, nchunk // NBUF, gg_body, 0)
        for k in range(NBUF - 1):
            g = nchunk - (NBUF - 1) + k
            wait_out(g, g % NBUF)

    grid_kernel = pl.kernel(
        body,
        out_type=jax.ShapeDtypeStruct((total, dim), jnp.float32),
        mesh=plsc.VectorSubcoreMesh(core_axis_name="c", subcore_axis_name="s"),
        scratch_types=[
            pltpu.VMEM((rows_per_w // IDX_PER_DMA, IDX_PER_DMA), jnp.int32),
            pltpu.VMEM((nrep * seq, dim), jnp.float32),
            pltpu.VMEM((CHUNK, dim), jnp.float32),
            pltpu.VMEM((CHUNK, dim), jnp.float32),
            pltpu.VMEM((CHUNK, dim), jnp.float32),
            pltpu.VMEM((CHUNK, dim), jnp.float32),
            pltpu.SemaphoreType.DMA((NBUF,)),
            pltpu.SemaphoreType.DMA((NBUF,)),
        ],
        compiler_params=pltpu.CompilerParams(use_tc_tiling_on_sc=False),
    )
    return grid_kernel(idx3, token_table, pos_table)


def kernel(inputs, token_table, pos_table):
    batch, seq = inputs.shape
    _, dim = token_table.shape
    total = batch * seq
    rows_per_w = total // NW
    idx3 = inputs.astype(jnp.int32).reshape(
        NW, rows_per_w // IDX_PER_DMA, IDX_PER_DMA)
    out = _embed(idx3, token_table, pos_table,
                 seq=seq, dim=dim, rows_per_w=rows_per_w)
    return out.reshape(batch, seq, dim)
